# baseline (speedup 1.0000x reference)
import functools

import jax
import jax.numpy as jnp
from jax import lax
from jax.experimental import pallas as pl
from jax.experimental.pallas import tpu as pltpu
from jax.experimental.pallas import tpu_sc as plsc

NUM_GATES = 8
CAPACITY_FACTOR = 1.25
MIN_EXPERT_CAPACITY = 4
CHUNK = 32
NC, NS = 2, 16


def _sc_body(ebv_hbm, out_d, out_c, ebv, obuf, sem, *, cap, gs, b):
    pltpu.sync_copy(ebv_hbm, ebv)

    z16 = jnp.zeros((16,), jnp.float32)

    def _zero_body(i, carry):
        ri = i // (NUM_GATES * (cap // 16))
        rem = i % (NUM_GATES * (cap // 16))
        ei = rem // (cap // 16)
        ci = (rem % (cap // 16)) * 16
        obuf[ri, ei, pl.ds(ci, 16)] = z16
        return carry

    lax.fori_loop(0, CHUNK * NUM_GATES * (cap // 16), _zero_body, 0)

    wid = lax.axis_index("s") * NC + lax.axis_index("c")

    rows_per_out = b * gs
    chunks_per_out = rows_per_out // CHUNK
    n_chunks = 2 * chunks_per_out
    per_worker = n_chunks // (NC * NS)

    iota16 = lax.broadcasted_iota(jnp.int32, (16,), 0)

    for j in range(per_worker):
        chunk = wid * per_worker + j
        out_sel = chunk // chunks_per_out
        rem = chunk % chunks_per_out
        bb = rem // (gs // CHUNK)
        t0 = (rem % (gs // CHUNK)) * CHUNK
        is_ones = t0 < cap

        @pl.when(is_ones)
        def _ones_case():
            evec = ebv[bb]
            for ep in range(NUM_GATES):
                gate_hit = evec == ep
                for i in range(CHUNK):
                    lane = i % 16
                    cstart = t0 + 16 * (i // 16)
                    val = jnp.where(gate_hit & (iota16 == lane),
                                    1.0, 0.0).astype(jnp.float32)
                    obuf[i, ep, pl.ds(cstart, 16)] = val

            @pl.when(out_sel == 0)
            def _():
                pltpu.async_copy(obuf, out_d.at[bb, pl.ds(t0, CHUNK)], sem).wait()

            @pl.when(out_sel == 1)
            def _():
                pltpu.async_copy(obuf, out_c.at[bb, pl.ds(t0, CHUNK)], sem).wait()

            for ep in range(NUM_GATES):
                for i in range(CHUNK):
                    cstart = t0 + 16 * (i // 16)
                    obuf[i, ep, pl.ds(cstart, 16)] = z16

        @pl.when(jnp.logical_not(is_ones))
        def _zero_case():
            @pl.when(out_sel == 0)
            def _():
                pltpu.async_copy(obuf, out_d.at[bb, pl.ds(t0, CHUNK)], sem).wait()

            @pl.when(out_sel == 1)
            def _():
                pltpu.async_copy(obuf, out_c.at[bb, pl.ds(t0, CHUNK)], sem).wait()


def kernel(x, current_y):
    b, gs, _ = x.shape
    cap = int(gs * CAPACITY_FACTOR / NUM_GATES)
    cap = max(min(gs, cap), MIN_EXPERT_CAPACITY)

    eb = jnp.remainder(current_y.astype(jnp.int32), NUM_GATES)
    eb16 = jnp.tile(eb[:, None], (1, 16))

    body = functools.partial(_sc_body, cap=cap, gs=gs, b=b)
    out_t = [
        jax.ShapeDtypeStruct((b, gs, NUM_GATES, cap), jnp.float32),
        jax.ShapeDtypeStruct((b, gs, NUM_GATES, cap), jnp.float32),
    ]
    mesh = plsc.VectorSubcoreMesh(core_axis_name="c", subcore_axis_name="s")
    dispatch, combine = pl.kernel(
        body,
        out_type=out_t,
        mesh=mesh,
        scratch_types=[
            pltpu.VMEM((b, 16), jnp.int32),
            pltpu.VMEM((CHUNK, NUM_GATES, cap), jnp.float32),
            pltpu.SemaphoreType.DMA,
        ],
    )(eb16)
    return dispatch, combine


# speedup vs baseline: 1.2275x; 1.2275x over previous
import functools

import jax
import jax.numpy as jnp
from jax import lax
from jax.experimental import pallas as pl
from jax.experimental.pallas import tpu as pltpu
from jax.experimental.pallas import tpu_sc as plsc

NUM_GATES = 8
CAPACITY_FACTOR = 1.25
MIN_EXPERT_CAPACITY = 4
CHUNK = 32
NC, NS = 2, 16


def _sc_body(ebv_hbm, out_d, out_c, ebv, obuf, sem, *, cap, gs, b):
    pltpu.sync_copy(ebv_hbm, ebv)

    z16 = jnp.zeros((16,), jnp.float32)
    nc16 = cap // 16

    def _zero_row(ri, carry):
        for ei in range(NUM_GATES):
            for c in range(nc16):
                obuf[ri, ei, pl.ds(c * 16, 16)] = z16
        return carry

    lax.fori_loop(0, CHUNK, _zero_row, 0)

    wid = lax.axis_index("s") * NC + lax.axis_index("c")

    n_rc = (b * gs) // CHUNK
    per_worker = n_rc // (NC * NS)
    rc_per_batch = gs // CHUNK

    iota16 = lax.broadcasted_iota(jnp.int32, (16,), 0)

    for j in range(per_worker):
        rc = j * (NC * NS) + wid
        bb = rc // rc_per_batch
        t0 = (rc % rc_per_batch) * CHUNK
        is_ones = t0 < cap

        @pl.when(is_ones)
        def _ones_case():
            evec = ebv[bb]
            for ep in range(NUM_GATES):
                gate_hit = evec == ep
                for i in range(CHUNK):
                    lane = i % 16
                    cstart = t0 + 16 * (i // 16)
                    val = jnp.where(gate_hit & (iota16 == lane),
                                    1.0, 0.0).astype(jnp.float32)
                    obuf[i, ep, pl.ds(cstart, 16)] = val

            c1 = pltpu.async_copy(obuf, out_d.at[bb, pl.ds(t0, CHUNK)], sem)
            c2 = pltpu.async_copy(obuf, out_c.at[bb, pl.ds(t0, CHUNK)], sem)
            c1.wait()
            c2.wait()

            for ep in range(NUM_GATES):
                for i in range(CHUNK):
                    cstart = t0 + 16 * (i // 16)
                    obuf[i, ep, pl.ds(cstart, 16)] = z16

        @pl.when(jnp.logical_not(is_ones))
        def _zero_case():
            c1 = pltpu.async_copy(obuf, out_d.at[bb, pl.ds(t0, CHUNK)], sem)
            c2 = pltpu.async_copy(obuf, out_c.at[bb, pl.ds(t0, CHUNK)], sem)
            c1.wait()
            c2.wait()


def kernel(x, current_y):
    b, gs, _ = x.shape
    cap = int(gs * CAPACITY_FACTOR / NUM_GATES)
    cap = max(min(gs, cap), MIN_EXPERT_CAPACITY)

    eb = jnp.remainder(current_y.astype(jnp.int32), NUM_GATES)
    eb16 = jnp.tile(eb[:, None], (1, 16))

    body = functools.partial(_sc_body, cap=cap, gs=gs, b=b)
    out_t = [
        jax.ShapeDtypeStruct((b, gs, NUM_GATES, cap), jnp.float32),
        jax.ShapeDtypeStruct((b, gs, NUM_GATES, cap), jnp.float32),
    ]
    mesh = plsc.VectorSubcoreMesh(core_axis_name="c", subcore_axis_name="s")
    dispatch, combine = pl.kernel(
        body,
        out_type=out_t,
        mesh=mesh,
        scratch_types=[
            pltpu.VMEM((b, 16), jnp.int32),
            pltpu.VMEM((CHUNK, NUM_GATES, cap), jnp.float32),
            pltpu.SemaphoreType.DMA,
        ],
    )(eb16)
    return dispatch, combine


# speedup vs baseline: 2.0995x; 1.7103x over previous
import functools

import jax
import jax.numpy as jnp
from jax.experimental import pallas as pl
from jax.experimental.pallas import tpu as pltpu

NUM_GATES = 8
CAPACITY_FACTOR = 1.25
MIN_EXPERT_CAPACITY = 4
TBLK = 1024


def _route_kernel(eb_ref, out_ref, *, cap, k_total):
    b = pl.program_id(0)
    tb = pl.program_id(1)
    e = eb_ref[b]
    t0 = tb * TBLK
    t = jax.lax.broadcasted_iota(jnp.int32, (TBLK, k_total), 0) + t0
    k = jax.lax.broadcasted_iota(jnp.int32, (TBLK, k_total), 1)
    val = jnp.where((t < cap) & (k == e * cap + t), 1.0, 0.0).astype(jnp.float32)
    out_ref[0] = val


def kernel(x, current_y):
    b, gs, _ = x.shape
    cap = int(gs * CAPACITY_FACTOR / NUM_GATES)
    cap = max(min(gs, cap), MIN_EXPERT_CAPACITY)
    k_total = NUM_GATES * cap

    eb = jnp.remainder(current_y.astype(jnp.int32), NUM_GATES)

    kern = functools.partial(_route_kernel, cap=cap, k_total=k_total)
    grid_spec = pltpu.PrefetchScalarGridSpec(
        num_scalar_prefetch=1,
        grid=(b, gs // TBLK),
        in_specs=[],
        out_specs=[
            pl.BlockSpec((1, TBLK, k_total), lambda i, j, eb_ref: (i, j, 0)),
        ],
    )
    out_shape = [
        jax.ShapeDtypeStruct((b, gs, k_total), jnp.float32),
    ]
    (out,) = pl.pallas_call(
        kern, grid_spec=grid_spec, out_shape=out_shape,
        compiler_params=pltpu.CompilerParams(
            dimension_semantics=("parallel", "parallel")),
    )(eb)
    out = out.reshape(b, gs, NUM_GATES, cap)
    return out, out
